# baseline (speedup 1.0000x reference)
import functools

import jax
import jax.numpy as jnp
from jax import lax
from jax.experimental import pallas as pl
from jax.experimental.pallas import tpu as pltpu
from jax.experimental.pallas import tpu_sc as plsc

_HI = lax.Precision.HIGHEST

_N = 10000
_E = 320000
_G = 64
_F = 96
_D2 = 192
_DT = 256
_NC = 2
_NS = 16
_NW = _NC * _NS
_EPW = _E // _NW
_EB = 40
_NBLK = _EPW // _EB
_NPAD = 10240
_RPT = _NPAD // _NS
_FP = 128


def _leaky(v):
    return jnp.where(v >= 0, v, 0.01 * v)


def _prologue_body(h0_ref, ax_ref, w1, b1, w2, b2, w3, b3, wa, ba,
                   wd, bd, wsc, sa_ref, pd_ref, ps_ref):
    h = h0_ref[...]
    h = _leaky(jnp.dot(h, w1[...]) + b1[...])
    h = _leaky(jnp.dot(h, w2[...]) + b2[...])
    h = _leaky(jnp.dot(h, w3[...]) + b3[...])
    a = jnp.dot(ax_ref[...], wa[...]) + ba[...]
    s = jnp.concatenate([h, a], axis=1)
    sa_ref[...] = s
    pd_ref[...] = jnp.dot(s, wd[...]) + bd[...]
    ps_ref[...] = jnp.dot(s, wsc[...])


def _prologue(h0, ax, w1, b1, w2, b2, w3, b3, wa, ba, wd, bd, wsc):
    R = 2000
    n = h0.shape[0]
    full = lambda shp: pl.BlockSpec(shp, lambda i: (0, 0))
    row = lambda width: pl.BlockSpec((R, width), lambda i: (i, 0))
    return pl.pallas_call(
        _prologue_body,
        grid=(n // R,),
        in_specs=[
            row(128), row(8),
            full((128, 128)), full((1, 128)),
            full((128, 128)), full((1, 128)),
            full((128, 64)), full((1, 64)),
            full((8, 32)), full((1, 32)),
            full((_F, _DT)), full((1, _DT)),
            full((_F, _DT)),
        ],
        out_specs=[row(_F), row(_DT), row(_DT)],
        out_shape=[
            jax.ShapeDtypeStruct((n, _F), jnp.float32),
            jax.ShapeDtypeStruct((n, _DT), jnp.float32),
            jax.ShapeDtypeStruct((n, _DT), jnp.float32),
        ],
    )(h0, ax, w1, b1, w2, b2, w3, b3, wa, ba, wd, bd, wsc)


def _ea_body(el_ref, wa1, wa2, wa3, o1, o2, o3):
    d = el_ref[...]
    mu = lax.broadcasted_iota(jnp.int32, (1, 32), 1).astype(jnp.float32) * 0.2
    t = d - mu
    g = jnp.exp(t * t * (-25.0))
    o1[...] = jnp.dot(g, wa1[...])
    o2[...] = jnp.dot(g, wa2[...])
    o3[...] = jnp.dot(g, wa3[...])


def _ea_tables(el2d, wa1, wa2, wa3):
    R = 2560
    e = el2d.shape[0]
    full = lambda shp: pl.BlockSpec(shp, lambda i: (0, 0))
    row = lambda width: pl.BlockSpec((R, width), lambda i: (i, 0))
    return pl.pallas_call(
        _ea_body,
        grid=(e // R,),
        in_specs=[row(1), full((32, _D2)), full((32, _D2)), full((32, _D2))],
        out_specs=[row(_D2), row(_D2), row(_D2)],
        out_shape=[jax.ShapeDtypeStruct((e, _D2), jnp.float32)] * 3,
    )(el2d, wa1, wa2, wa3)


def _prep_body(sa_ref, aa_ref, ab_ref, wd, bd, wsc, sa_out, pd_ref, ps_ref):
    s = _leaky(sa_ref[...] + aa_ref[...] + ab_ref[...])
    sa_out[...] = s
    pd_ref[...] = jnp.dot(s, wd[...]) + bd[...]
    ps_ref[...] = jnp.dot(s, wsc[...])


def _layer_prep(sa, acc_a, acc_b, wd, bd, wsc):
    R = 2000
    n = sa.shape[0]
    full = lambda shp: pl.BlockSpec(shp, lambda i: (0, 0))
    row = lambda width: pl.BlockSpec((R, width), lambda i: (i, 0))
    return pl.pallas_call(
        _prep_body,
        grid=(n // R,),
        in_specs=[row(_F), row(_F), row(_F),
                  full((_F, _DT)), full((1, _DT)), full((_F, _DT))],
        out_specs=[row(_F), row(_DT), row(_DT)],
        out_shape=[
            jax.ShapeDtypeStruct((n, _F), jnp.float32),
            jax.ShapeDtypeStruct((n, _DT), jnp.float32),
            jax.ShapeDtypeStruct((n, _DT), jnp.float32),
        ],
    )(sa, acc_a, acc_b, wd, bd, wsc)


def _sc_edge_body(src_hbm, dst_hbm, pd_hbm, ps_hbm, ea_hbm, z_hbm, out_hbm,
                  srcv, dstv, gdv, gsv, eav, mv, acc, sem1, sem2):
    c = lax.axis_index("c")
    s = lax.axis_index("s")
    wid = c * _NS + s

    pltpu.sync_copy(z_hbm.at[pl.ds(s * _RPT, _RPT)],
                    acc.at[pl.ds(s * _RPT, _RPT)])

    def zpad(e, _):
        for j in range(_F // 16, _FP // 16):
            mv[e, pl.ds(j * 16, 16)] = jnp.zeros((16,), jnp.float32)
        return 0

    lax.fori_loop(0, _EB, zpad, 0)
    plsc.subcore_barrier()

    def blk(i, _):
        base = wid * _EPW + i * _EB
        pltpu.sync_copy(src_hbm.at[pl.ds(base, _EB)], srcv)
        pltpu.sync_copy(dst_hbm.at[pl.ds(base, _EB)], dstv)
        pltpu.sync_copy(ea_hbm.at[pl.ds(base, _EB)], eav)
        gd = pltpu.async_copy(pd_hbm.at[dstv], gdv, sem1)
        gs = pltpu.async_copy(ps_hbm.at[srcv], gsv, sem2)
        gd.wait()
        gs.wait()

        def edge(e, _):
            for j in range(_F // 16):
                sla = pl.ds(j * 16, 16)
                slb = pl.ds(_F + j * 16, 16)
                af = gdv[e, sla] + gsv[e, sla] + eav[e, sla]
                bt = gdv[e, slb] + gsv[e, slb] + eav[e, slb]
                sig = 1.0 / (1.0 + jnp.exp(-af))
                t = jnp.exp(-jnp.abs(bt))
                sfr = t / (t + 2.0)
                z2 = sfr * sfr
                p = 1.0 + z2 * ((1.0 / 3.0) + z2 * (0.2 + z2 * (1.0 / 7.0)))
                sp = jnp.maximum(bt, 0.0) + (sfr + sfr) * p
                mv[e, sla] = sig * sp
            return 0

        lax.fori_loop(0, _EB, edge, 0)
        pltpu.sync_copy(mv, acc.at[dstv], add=True)
        return 0

    lax.fori_loop(0, _NBLK, blk, 0)
    plsc.subcore_barrier()
    pltpu.sync_copy(acc.at[pl.ds(s * _RPT, _RPT)],
                    out_hbm.at[c, pl.ds(s * _RPT, _RPT)])


@functools.cache
def _get_sc_edge():
    return pl.kernel(
        _sc_edge_body,
        out_type=jax.ShapeDtypeStruct((_NC, _NPAD, _FP), jnp.float32),
        mesh=plsc.VectorSubcoreMesh(core_axis_name="c", subcore_axis_name="s",
                                    num_cores=_NC, num_subcores=_NS),
        scratch_types=[
            pltpu.VMEM((_EB,), jnp.int32),
            pltpu.VMEM((_EB,), jnp.int32),
            pltpu.VMEM((_EB, _DT), jnp.float32),
            pltpu.VMEM((_EB, _DT), jnp.float32),
            pltpu.VMEM((_EB, _D2), jnp.float32),
            pltpu.VMEM((_EB, _FP), jnp.float32),
            pltpu.VMEM_SHARED((_NPAD, _FP), jnp.float32),
            pltpu.SemaphoreType.DMA,
            pltpu.SemaphoreType.DMA,
        ],
    )


def _final_body(sa_ref, aa_ref, ab_ref, seg_ref, wd1, bd1, wd2, bd2,
                out_ref, pooled):
    i = pl.program_id(0)

    @pl.when(i == 0)
    def _():
        pooled[...] = jnp.zeros_like(pooled)

    s = sa_ref[...] + aa_ref[...] + ab_ref[...]
    seg = seg_ref[...]
    oh = (seg == lax.broadcasted_iota(jnp.int32, (seg.shape[0], _G), 1))
    oh = oh.astype(jnp.float32)
    pooled[...] += lax.dot_general(oh, s, (((0,), (0,)), ((), ())),
                                   precision=_HI)

    @pl.when(i == pl.num_programs(0) - 1)
    def _():
        q = jnp.maximum(jnp.dot(pooled[...], wd1[...]) + bd1[...], 0.0)
        out_ref[...] = jnp.dot(q, wd2[...]) + bd2[...]


def _final(sa, acc_a, acc_b, seg2d, wd1, bd1, wd2, bd2):
    R = 2000
    n = sa.shape[0]
    full = lambda shp: pl.BlockSpec(shp, lambda i: (0, 0))
    row = lambda width: pl.BlockSpec((R, width), lambda i: (i, 0))
    return pl.pallas_call(
        _final_body,
        grid=(n // R,),
        in_specs=[row(_F), row(_F), row(_F), row(1),
                  full((_F, 64)), full((1, 64)), full((64, 1)), full((1, 1))],
        out_specs=full((_G, 1)),
        out_shape=jax.ShapeDtypeStruct((_G, 1), jnp.float32),
        scratch_shapes=[pltpu.VMEM((_G, _F), jnp.float32)],
    )(sa, acc_a, acc_b, seg2d, wd1, bd1, wd2, bd2)


def kernel(x, edge_index, edge_length, forces_stack, forces_norm, batch,
           action_x, W_e1, b_e1, W_e2, b_e2, W_e3, b_e3, W_a, b_a,
           Wf1, bf1, Ws1, bs1, Wf2, bf2, Ws2, bs2, Wf3, bf3, Ws3, bs3,
           W_d1, b_d1, W_d2, b_d2):
    f32 = jnp.float32
    h0 = jnp.concatenate([x, forces_stack, forces_norm], axis=1)
    ax = jnp.pad(action_x, ((0, 0), (0, 5)))
    wa = jnp.pad(W_a, ((0, 5), (0, 0)))

    def split(Wf, bf, Ws, bs):
        wd = jnp.concatenate([Wf[:_F], Ws[:_F]], axis=1)
        wd = jnp.pad(wd, ((0, 0), (0, _DT - _D2)))
        bd = jnp.pad(jnp.concatenate([bf, bs]).reshape(1, _D2),
                     ((0, 0), (0, _DT - _D2)))
        wsc = jnp.concatenate([Wf[_F:2 * _F], Ws[_F:2 * _F]], axis=1)
        wsc = jnp.pad(wsc, ((0, 0), (0, _DT - _D2)))
        wea = jnp.concatenate([Wf[2 * _F:], Ws[2 * _F:]], axis=1)
        wea = jnp.pad(wea, ((0, 6), (0, 0)))
        return wd, bd, wsc, wea

    wd1, bd1, wsc1, wea1 = split(Wf1, bf1, Ws1, bs1)
    wd2, bd2, wsc2, wea2 = split(Wf2, bf2, Ws2, bs2)
    wd3, bd3, wsc3, wea3 = split(Wf3, bf3, Ws3, bs3)

    sa0, pd, ps = _prologue(h0, ax, W_e1, b_e1.reshape(1, -1),
                            W_e2, b_e2.reshape(1, -1),
                            W_e3, b_e3.reshape(1, -1), wa,
                            b_a.reshape(1, -1), wd1, bd1, wsc1)

    ea1, ea2, ea3 = _ea_tables(edge_length.reshape(_E, 1), wea1, wea2, wea3)

    src = edge_index[0]
    dst = edge_index[1]
    z0 = jnp.zeros((_NPAD, _FP), f32)

    acc = _get_sc_edge()(src, dst, pd, ps, ea1, z0)
    sa1, pd, ps = _layer_prep(sa0, acc[0, :_N, :_F], acc[1, :_N, :_F],
                              wd2, bd2, wsc2)
    acc = _get_sc_edge()(src, dst, pd, ps, ea2, z0)
    sa2, pd, ps = _layer_prep(sa1, acc[0, :_N, :_F], acc[1, :_N, :_F],
                              wd3, bd3, wsc3)
    acc = _get_sc_edge()(src, dst, pd, ps, ea3, z0)

    y = _final(sa2, acc[0, :_N, :_F], acc[1, :_N, :_F],
               batch.reshape(_N, 1).astype(jnp.int32),
               W_d1, b_d1.reshape(1, -1), W_d2, b_d2.reshape(1, -1))
    return y[:, 0]
